# baseline (speedup 1.0000x reference)
import functools

import jax
import jax.numpy as jnp
from jax import lax
from jax.experimental import pallas as pl
from jax.experimental.pallas import tpu as pltpu
from jax.experimental.pallas import tpu_sc as plsc

BATCH = 16384
FACTOR = 32
ROWS_PER_GROUP = 128 // FACTOR
NC = 2
NS = 16
NW = NC * NS
B_PER_W = BATCH // NW
CHUNK = 128
NCHUNK = B_PER_W // CHUNK
NGRP = CHUNK // 16


def _gmf_body(uidx_hbm, iidx_hbm, utab_hbm, itab_hbm, out_hbm,
              uidx_v, iidx_v, ushf_v, ishf_v,
              ubuf0, ubuf1, ibuf0, ibuf1, out_v, sem0, sem1):
    wid = lax.axis_index("s") * NC + lax.axis_index("c")
    blk = wid * NCHUNK

    pltpu.sync_copy(uidx_hbm.at[pl.ds(blk, NCHUNK), :], uidx_v)
    pltpu.sync_copy(iidx_hbm.at[pl.ds(blk, NCHUNK), :], iidx_v)

    for j in range(NCHUNK):
        for g in range(NGRP):
            s = pl.ds(g * 16, 16)
            ushf_v[j, s] = lax.shift_right_logical(uidx_v[j, s], 2)
            ishf_v[j, s] = lax.shift_right_logical(iidx_v[j, s], 2)

    ubufs = (ubuf0, ubuf1)
    ibufs = (ibuf0, ibuf1)
    sems = (sem0, sem1)

    def start(j):
        return (
            pltpu.async_copy(utab_hbm.at[ushf_v.at[j]], ubufs[j % 2], sems[j % 2]),
            pltpu.async_copy(itab_hbm.at[ishf_v.at[j]], ibufs[j % 2], sems[j % 2]),
        )

    pending = [start(0), start(1)]

    for j in range(NCHUNK):
        for c in pending[j]:
            c.wait()
        ub, ib = ubufs[j % 2], ibufs[j % 2]

        def group_body(g, _, j=j, ub=ub, ib=ib):
            s = pl.ds(g * 16, 16)
            riota = lax.iota(jnp.int32, 16)
            rows = riota + g * 16
            orow_hi = lax.shift_right_logical(rows, 2) + j * (CHUNK // 4)
            ocol = lax.shift_left(lax.bitwise_and(riota, 3), 5)
            uoff = lax.shift_left(lax.bitwise_and(uidx_v[j, s], 3), 5)
            ioff = lax.shift_left(lax.bitwise_and(iidx_v[j, s], 3), 5)
            for c0 in range(FACTOR):
                uval = plsc.load_gather(ub, [rows, uoff + c0])
                ival = plsc.load_gather(ib, [rows, ioff + c0])
                plsc.store_scatter(out_v, [orow_hi, ocol + c0], uval * ival)
            return ()

        lax.fori_loop(0, NGRP, group_body, ())

        if j + 2 < NCHUNK:
            pending.append(start(j + 2))

    pltpu.sync_copy(out_v, out_hbm.at[pl.ds(wid * (B_PER_W // 4), B_PER_W // 4), :])


@jax.jit
def _gmf(uidx2, iidx2, utab2, itab2):
    mesh = plsc.VectorSubcoreMesh(core_axis_name="c", subcore_axis_name="s")
    kfn = functools.partial(
        pl.kernel,
        mesh=mesh,
        compiler_params=pltpu.CompilerParams(needs_layout_passes=False),
        out_type=jax.ShapeDtypeStruct((BATCH // 4, 128), jnp.float32),
        scratch_types=[
            pltpu.VMEM((NCHUNK, CHUNK), jnp.int32),
            pltpu.VMEM((NCHUNK, CHUNK), jnp.int32),
            pltpu.VMEM((NCHUNK, CHUNK), jnp.int32),
            pltpu.VMEM((NCHUNK, CHUNK), jnp.int32),
            pltpu.VMEM((CHUNK, 128), jnp.float32),
            pltpu.VMEM((CHUNK, 128), jnp.float32),
            pltpu.VMEM((CHUNK, 128), jnp.float32),
            pltpu.VMEM((CHUNK, 128), jnp.float32),
            pltpu.VMEM((B_PER_W // 4, 128), jnp.float32),
            pltpu.SemaphoreType.DMA,
            pltpu.SemaphoreType.DMA,
        ],
    )(_gmf_body)
    return kfn(uidx2, iidx2, utab2, itab2)


def kernel(user_indices, item_indices, user_table, item_table):
    uidx2 = user_indices.astype(jnp.int32).reshape(BATCH // CHUNK, CHUNK)
    iidx2 = item_indices.astype(jnp.int32).reshape(BATCH // CHUNK, CHUNK)
    utab2 = user_table.reshape(-1, 128)
    itab2 = item_table.reshape(-1, 128)
    out = _gmf(uidx2, iidx2, utab2, itab2)
    return out.reshape(BATCH, FACTOR)


# speedup vs baseline: 3.7402x; 3.7402x over previous
import functools

import jax
import jax.numpy as jnp
from jax import lax
from jax.experimental import pallas as pl
from jax.experimental.pallas import tpu as pltpu
from jax.experimental.pallas import tpu_sc as plsc

BATCH = 16384
FACTOR = 32
TABLE = 1000000
NC = 2
NS = 16
NW = NC * NS
B_PER_W = BATCH // NW
CHUNK = 128
NCHUNK = B_PER_W // CHUNK
NRING = 4


def _gmf_body(uidx_hbm, iidx_hbm, utab_hbm, itab_hbm, out_hbm,
              uidx_v, iidx_v, uring, iring, out_v, usem, isem):
    wid = lax.axis_index("s") * NC + lax.axis_index("c")
    base = wid * B_PER_W

    pltpu.sync_copy(uidx_hbm.at[pl.ds(wid * NCHUNK, NCHUNK), :], uidx_v)
    pltpu.sync_copy(iidx_hbm.at[pl.ds(wid * NCHUNK, NCHUNK), :], iidx_v)

    iota = lax.iota(jnp.int32, 16)

    def fetch(r, ring, tab, sem, slot):
        blk = pl.multiple_of(lax.bitwise_and(r, ~127), 128)
        pltpu.async_copy(tab.at[:, pl.ds(blk, CHUNK)],
                         ring.at[slot], sem.at[slot])

    def drain(ring, tab, sem, slot):
        pltpu.make_async_copy(tab.at[:, pl.ds(0, CHUNK)],
                              ring.at[slot], sem.at[slot]).wait()

    def extract(ring, slot, rmod):
        cols = jnp.broadcast_to(rmod, (16,))
        lo = plsc.load_gather(ring.at[slot], [iota, cols])
        hi = plsc.load_gather(ring.at[slot], [iota + 16, cols])
        return lo, hi

    def group_body(g, _):
        j = lax.shift_right_logical(g, 3)
        cb = lax.shift_left(lax.bitwise_and(g, 7), 4)
        uv = uidx_v[j, pl.ds(cb, 16)]
        iv = iidx_v[j, pl.ds(cb, 16)]
        b0 = lax.shift_left(g, 4)

        for l in range(NRING):
            fetch(uv[l], uring, utab_hbm, usem, l)
            fetch(iv[l], iring, itab_hbm, isem, l)
        for l in range(16):
            slot = l % NRING
            drain(uring, utab_hbm, usem, slot)
            drain(iring, itab_hbm, isem, slot)
            um = lax.bitwise_and(uv[l], 127)
            im = lax.bitwise_and(iv[l], 127)
            ulo, uhi = extract(uring, slot, um)
            ilo, ihi = extract(iring, slot, im)
            bcol = jnp.broadcast_to(b0 + l, (16,))
            plsc.store_scatter(out_v, [iota, bcol], ulo * ilo)
            plsc.store_scatter(out_v, [iota + 16, bcol], uhi * ihi)
            if l + NRING < 16:
                fetch(uv[l + NRING], uring, utab_hbm, usem, slot)
                fetch(iv[l + NRING], iring, itab_hbm, isem, slot)
        return ()

    lax.fori_loop(0, B_PER_W // 16, group_body, ())

    pltpu.sync_copy(out_v, out_hbm.at[:, pl.ds(base, B_PER_W)])


@jax.jit
def _gmf(uidx2, iidx2, utabT, itabT):
    mesh = plsc.VectorSubcoreMesh(core_axis_name="c", subcore_axis_name="s")
    kfn = functools.partial(
        pl.kernel,
        mesh=mesh,
        compiler_params=pltpu.CompilerParams(needs_layout_passes=False),
        out_type=jax.ShapeDtypeStruct((FACTOR, BATCH), jnp.float32),
        scratch_types=[
            pltpu.VMEM((NCHUNK, CHUNK), jnp.int32),
            pltpu.VMEM((NCHUNK, CHUNK), jnp.int32),
            pltpu.VMEM((NRING, FACTOR, CHUNK), jnp.float32),
            pltpu.VMEM((NRING, FACTOR, CHUNK), jnp.float32),
            pltpu.VMEM((FACTOR, B_PER_W), jnp.float32),
            pltpu.SemaphoreType.DMA((NRING,)),
            pltpu.SemaphoreType.DMA((NRING,)),
        ],
    )(_gmf_body)
    return kfn(uidx2, iidx2, utabT, itabT)


def kernel(user_indices, item_indices, user_table, item_table):
    uidx2 = user_indices.astype(jnp.int32).reshape(BATCH // CHUNK, CHUNK)
    iidx2 = item_indices.astype(jnp.int32).reshape(BATCH // CHUNK, CHUNK)
    outT = _gmf(uidx2, iidx2, user_table.T, item_table.T)
    return outT.T


# speedup vs baseline: 3.7573x; 1.0046x over previous
import functools

import jax
import jax.numpy as jnp
from jax import lax
from jax.experimental import pallas as pl
from jax.experimental.pallas import tpu as pltpu
from jax.experimental.pallas import tpu_sc as plsc

BATCH = 16384
FACTOR = 32
TABLE = 1000000
NC = 2
NS = 16
NW = NC * NS
B_PER_W = BATCH // NW
CHUNK = 128
NCHUNK = B_PER_W // CHUNK
NRING = 8


def _gmf_body(uidx_hbm, iidx_hbm, utab_hbm, itab_hbm, out_hbm,
              uidx_v, iidx_v, uring, iring, out_v, usem, isem):
    wid = lax.axis_index("s") * NC + lax.axis_index("c")
    base = wid * B_PER_W

    pltpu.sync_copy(uidx_hbm.at[pl.ds(wid * NCHUNK, NCHUNK), :], uidx_v)
    pltpu.sync_copy(iidx_hbm.at[pl.ds(wid * NCHUNK, NCHUNK), :], iidx_v)

    iota = lax.iota(jnp.int32, 16)

    def fetch(r, ring, tab, sem, slot):
        blk = pl.multiple_of(lax.bitwise_and(r, ~127), 128)
        pltpu.async_copy(tab.at[:, pl.ds(blk, CHUNK)],
                         ring.at[slot], sem.at[slot])

    def drain(ring, tab, sem, slot):
        pltpu.make_async_copy(tab.at[:, pl.ds(0, CHUNK)],
                              ring.at[slot], sem.at[slot]).wait()

    def extract(ring, slot, rmod):
        cols = jnp.broadcast_to(rmod, (16,))
        lo = plsc.load_gather(ring.at[slot], [iota, cols])
        hi = plsc.load_gather(ring.at[slot], [iota + 16, cols])
        return lo, hi

    def group_body(g, _):
        j = lax.shift_right_logical(g, 3)
        cb = lax.shift_left(lax.bitwise_and(g, 7), 4)
        uv = uidx_v[j, pl.ds(cb, 16)]
        iv = iidx_v[j, pl.ds(cb, 16)]
        b0 = lax.shift_left(g, 4)

        for l in range(NRING):
            fetch(uv[l], uring, utab_hbm, usem, l)
            fetch(iv[l], iring, itab_hbm, isem, l)
        for l in range(16):
            slot = l % NRING
            drain(uring, utab_hbm, usem, slot)
            drain(iring, itab_hbm, isem, slot)
            um = lax.bitwise_and(uv[l], 127)
            im = lax.bitwise_and(iv[l], 127)
            ulo, uhi = extract(uring, slot, um)
            ilo, ihi = extract(iring, slot, im)
            bcol = jnp.broadcast_to(b0 + l, (16,))
            plsc.store_scatter(out_v, [iota, bcol], ulo * ilo)
            plsc.store_scatter(out_v, [iota + 16, bcol], uhi * ihi)
            if l + NRING < 16:
                fetch(uv[l + NRING], uring, utab_hbm, usem, slot)
                fetch(iv[l + NRING], iring, itab_hbm, isem, slot)
        return ()

    lax.fori_loop(0, B_PER_W // 16, group_body, ())

    pltpu.sync_copy(out_v, out_hbm.at[:, pl.ds(base, B_PER_W)])


@jax.jit
def _gmf(uidx2, iidx2, utabT, itabT):
    mesh = plsc.VectorSubcoreMesh(core_axis_name="c", subcore_axis_name="s")
    kfn = functools.partial(
        pl.kernel,
        mesh=mesh,
        compiler_params=pltpu.CompilerParams(needs_layout_passes=False),
        out_type=jax.ShapeDtypeStruct((FACTOR, BATCH), jnp.float32),
        scratch_types=[
            pltpu.VMEM((NCHUNK, CHUNK), jnp.int32),
            pltpu.VMEM((NCHUNK, CHUNK), jnp.int32),
            pltpu.VMEM((NRING, FACTOR, CHUNK), jnp.float32),
            pltpu.VMEM((NRING, FACTOR, CHUNK), jnp.float32),
            pltpu.VMEM((FACTOR, B_PER_W), jnp.float32),
            pltpu.SemaphoreType.DMA((NRING,)),
            pltpu.SemaphoreType.DMA((NRING,)),
        ],
    )(_gmf_body)
    return kfn(uidx2, iidx2, utabT, itabT)


def kernel(user_indices, item_indices, user_table, item_table):
    uidx2 = user_indices.astype(jnp.int32).reshape(BATCH // CHUNK, CHUNK)
    iidx2 = item_indices.astype(jnp.int32).reshape(BATCH // CHUNK, CHUNK)
    outT = _gmf(uidx2, iidx2, user_table.T, item_table.T)
    return outT.T
